# baseline (speedup 1.0000x reference)
import functools

import jax
import jax.numpy as jnp
from jax import lax
from jax.experimental import pallas as pl
from jax.experimental.pallas import tpu as pltpu
from jax.experimental.pallas import tpu_sc as plsc

NC = 2
NS = 16
L = 16
NW = NC * NS
B = 16384
F = 128
BPW = B // NW
CHUNK = 128
NCHUNK = BPW // CHUNK

_mesh = plsc.VectorSubcoreMesh(
    core_axis_name="c", subcore_axis_name="s", num_cores=NC, num_subcores=NS
)


@functools.partial(
    pl.kernel,
    mesh=_mesh,
    out_type=jax.ShapeDtypeStruct((B,), jnp.float32),
    compiler_params=pltpu.CompilerParams(needs_layout_passes=False),
    scratch_types=[
        pltpu.VMEM((NCHUNK, CHUNK), jnp.int32),
        pltpu.VMEM((NCHUNK, CHUNK), jnp.int32),
        pltpu.VMEM((CHUNK, F), jnp.float32),
        pltpu.VMEM((CHUNK, F), jnp.float32),
        pltpu.VMEM((CHUNK, F), jnp.float32),
        pltpu.VMEM((CHUNK, F), jnp.float32),
        pltpu.VMEM((BPW,), jnp.float32),
        pltpu.SemaphoreType.DMA,
        pltpu.SemaphoreType.DMA,
        pltpu.SemaphoreType.DMA,
        pltpu.SemaphoreType.DMA,
    ],
)
def _mf_kernel(user_hbm, item_hbm, uf_hbm, if_hbm, out_hbm,
               uidx, iidx, urows0, urows1, vrows0, vrows1, outv,
               semu0, semu1, semv0, semv1):
    wid = lax.axis_index("s") * NC + lax.axis_index("c")
    base = wid * BPW
    pltpu.sync_copy(user_hbm.at[wid], uidx)
    pltpu.sync_copy(item_hbm.at[wid], iidx)

    iota = lax.iota(jnp.int32, L)
    ubufs = (urows0, urows1)
    vbufs = (vrows0, vrows1)
    usems = (semu0, semu1)
    vsems = (semv0, semv1)

    def start(c):
        b = c % 2
        cu = pltpu.async_copy(uf_hbm.at[uidx.at[c]], ubufs[b], usems[b])
        cv = pltpu.async_copy(if_hbm.at[iidx.at[c]], vbufs[b], vsems[b])
        return cu, cv

    handles = {0: start(0)}
    for c in range(NCHUNK):
        if c + 1 < NCHUNK:
            handles[(c + 1) % 2] = start(c + 1)
        cu, cv = handles[c % 2]
        cu.wait()
        cv.wait()
        urows = ubufs[c % 2]
        vrows = vbufs[c % 2]

        def do_group(g, carry2, urows=urows, vrows=vrows, c=c):
            base_r = g * L
            vec = jnp.zeros((L,), jnp.float32)
            for i in range(L):
                r = base_r + i
                acc = urows[r, pl.ds(0, L)] * vrows[r, pl.ds(0, L)]
                for j in range(1, F // L):
                    acc = acc + urows[r, pl.ds(j * L, L)] * vrows[r, pl.ds(j * L, L)]
                vec = jnp.where(iota == i, jnp.sum(acc), vec)
            outv[pl.ds(c * CHUNK + base_r, L)] = vec
            return carry2

        lax.fori_loop(0, CHUNK // L, do_group, 0)

    pltpu.sync_copy(outv, out_hbm.at[pl.ds(base, BPW)])


def kernel(user, item, user_factors, item_factors):
    u3 = user.reshape(NW, NCHUNK, CHUNK).astype(jnp.int32)
    i3 = item.reshape(NW, NCHUNK, CHUNK).astype(jnp.int32)
    return _mf_kernel(u3, i3, user_factors, item_factors)


# speedup vs baseline: 1.3962x; 1.3962x over previous
import functools

import jax
import jax.numpy as jnp
from jax import lax
from jax.experimental import pallas as pl
from jax.experimental.pallas import tpu as pltpu
from jax.experimental.pallas import tpu_sc as plsc

NC = 2
NS = 16
L = 16
NW = NC * NS
B = 16384
F = 128
BPW = B // NW
CHUNK = 128
NCHUNK = BPW // CHUNK

_mesh = plsc.VectorSubcoreMesh(
    core_axis_name="c", subcore_axis_name="s", num_cores=NC, num_subcores=NS
)


@functools.partial(
    pl.kernel,
    mesh=_mesh,
    out_type=jax.ShapeDtypeStruct((B,), jnp.float32),
    compiler_params=pltpu.CompilerParams(needs_layout_passes=False),
    scratch_types=[
        pltpu.VMEM((NCHUNK, CHUNK), jnp.int32),
        pltpu.VMEM((NCHUNK, CHUNK), jnp.int32),
        pltpu.VMEM((CHUNK, F), jnp.float32),
        pltpu.VMEM((CHUNK, F), jnp.float32),
        pltpu.VMEM((CHUNK, F), jnp.float32),
        pltpu.VMEM((CHUNK, F), jnp.float32),
        pltpu.VMEM((CHUNK * L,), jnp.float32),
        pltpu.VMEM((BPW,), jnp.float32),
        pltpu.SemaphoreType.DMA,
        pltpu.SemaphoreType.DMA,
        pltpu.SemaphoreType.DMA,
        pltpu.SemaphoreType.DMA,
    ],
)
def _mf_kernel(user_hbm, item_hbm, uf_hbm, if_hbm, out_hbm,
               uidx, iidx, urows0, urows1, vrows0, vrows1, part, outv,
               semu0, semu1, semv0, semv1):
    wid = lax.axis_index("s") * NC + lax.axis_index("c")
    base = wid * BPW
    pltpu.sync_copy(user_hbm.at[wid], uidx)
    pltpu.sync_copy(item_hbm.at[wid], iidx)

    iota = lax.iota(jnp.int32, L)
    ubufs = (urows0, urows1)
    vbufs = (vrows0, vrows1)
    usems = (semu0, semu1)
    vsems = (semv0, semv1)

    def start(c):
        b = c % 2
        cu = pltpu.async_copy(uf_hbm.at[uidx.at[c]], ubufs[b], usems[b])
        cv = pltpu.async_copy(if_hbm.at[iidx.at[c]], vbufs[b], vsems[b])
        return cu, cv

    handles = {0: start(0)}
    for c in range(NCHUNK):
        if c + 1 < NCHUNK:
            handles[(c + 1) % 2] = start(c + 1)
        cu, cv = handles[c % 2]
        cu.wait()
        cv.wait()
        urows = ubufs[c % 2]
        vrows = vbufs[c % 2]

        def do_row(r, carry2, urows=urows, vrows=vrows):
            acc = urows[r, pl.ds(0, L)] * vrows[r, pl.ds(0, L)]
            for j in range(1, F // L):
                acc = acc + urows[r, pl.ds(j * L, L)] * vrows[r, pl.ds(j * L, L)]
            part[pl.ds(r * L, L)] = jnp.cumsum(acc)
            return carry2

        lax.fori_loop(0, CHUNK, do_row, 0)

        def do_group(g, carry3, c=c):
            s = plsc.load_gather(part, [g * (L * L) + iota * L + (L - 1)])
            outv[pl.ds(c * CHUNK + g * L, L)] = s
            return carry3

        lax.fori_loop(0, CHUNK // L, do_group, 0)

    pltpu.sync_copy(outv, out_hbm.at[pl.ds(base, BPW)])


def kernel(user, item, user_factors, item_factors):
    u3 = user.reshape(NW, NCHUNK, CHUNK).astype(jnp.int32)
    i3 = item.reshape(NW, NCHUNK, CHUNK).astype(jnp.int32)
    return _mf_kernel(u3, i3, user_factors, item_factors)


# speedup vs baseline: 1.4004x; 1.0030x over previous
import functools

import jax
import jax.numpy as jnp
from jax import lax
from jax.experimental import pallas as pl
from jax.experimental.pallas import tpu as pltpu
from jax.experimental.pallas import tpu_sc as plsc

NC = 2
NS = 16
L = 16
NW = NC * NS
B = 16384
F = 128
BPW = B // NW
CHUNK = 128
NCHUNK = BPW // CHUNK

_mesh = plsc.VectorSubcoreMesh(
    core_axis_name="c", subcore_axis_name="s", num_cores=NC, num_subcores=NS
)


@functools.partial(
    pl.kernel,
    mesh=_mesh,
    out_type=jax.ShapeDtypeStruct((B,), jnp.float32),
    compiler_params=pltpu.CompilerParams(needs_layout_passes=False),
    scratch_types=[
        pltpu.VMEM((BPW,), jnp.int32),
        pltpu.VMEM((BPW,), jnp.int32),
        pltpu.VMEM((CHUNK, F), jnp.float32),
        pltpu.VMEM((CHUNK, F), jnp.float32),
        pltpu.VMEM((CHUNK, F), jnp.float32),
        pltpu.VMEM((CHUNK, F), jnp.float32),
        pltpu.VMEM((CHUNK * L,), jnp.float32),
        pltpu.VMEM((BPW,), jnp.float32),
        pltpu.SemaphoreType.DMA,
        pltpu.SemaphoreType.DMA,
        pltpu.SemaphoreType.DMA,
        pltpu.SemaphoreType.DMA,
    ],
)
def _mf_kernel(user_hbm, item_hbm, uf_hbm, if_hbm, out_hbm,
               uidx, iidx, urows0, urows1, vrows0, vrows1, part, outv,
               semu0, semu1, semv0, semv1):
    wid = lax.axis_index("s") * NC + lax.axis_index("c")
    base = wid * BPW
    pltpu.sync_copy(user_hbm.at[pl.ds(base, BPW)], uidx)
    pltpu.sync_copy(item_hbm.at[pl.ds(base, BPW)], iidx)

    iota = lax.iota(jnp.int32, L)
    ubufs = (urows0, urows1)
    vbufs = (vrows0, vrows1)
    usems = (semu0, semu1)
    vsems = (semv0, semv1)

    def start(c):
        b = c % 2
        cu = pltpu.async_copy(uf_hbm.at[uidx.at[pl.ds(c * CHUNK, CHUNK)]], ubufs[b], usems[b])
        cv = pltpu.async_copy(if_hbm.at[iidx.at[pl.ds(c * CHUNK, CHUNK)]], vbufs[b], vsems[b])
        return cu, cv

    handles = {0: start(0)}
    for c in range(NCHUNK):
        if c + 1 < NCHUNK:
            handles[(c + 1) % 2] = start(c + 1)
        cu, cv = handles[c % 2]
        cu.wait()
        cv.wait()
        urows = ubufs[c % 2]
        vrows = vbufs[c % 2]

        def do_row(r, carry2, urows=urows, vrows=vrows):
            acc = urows[r, pl.ds(0, L)] * vrows[r, pl.ds(0, L)]
            for j in range(1, F // L):
                acc = acc + urows[r, pl.ds(j * L, L)] * vrows[r, pl.ds(j * L, L)]
            part[pl.ds(r * L, L)] = jnp.cumsum(acc)
            return carry2

        lax.fori_loop(0, CHUNK, do_row, 0)

        def do_group(g, carry3, c=c):
            s = plsc.load_gather(part, [g * (L * L) + iota * L + (L - 1)])
            outv[pl.ds(c * CHUNK + g * L, L)] = s
            return carry3

        lax.fori_loop(0, CHUNK // L, do_group, 0)

    pltpu.sync_copy(outv, out_hbm.at[pl.ds(base, BPW)])


def kernel(user, item, user_factors, item_factors):
    return _mf_kernel(user, item, user_factors, item_factors)


# speedup vs baseline: 1.4153x; 1.0107x over previous
import functools

import jax
import jax.numpy as jnp
from jax import lax
from jax.experimental import pallas as pl
from jax.experimental.pallas import tpu as pltpu
from jax.experimental.pallas import tpu_sc as plsc

NC = 2
NS = 16
L = 16
NW = NC * NS
B = 16384
F = 128
BPW = B // NW
CHUNK = 128
NCHUNK = BPW // CHUNK

_mesh = plsc.VectorSubcoreMesh(
    core_axis_name="c", subcore_axis_name="s", num_cores=NC, num_subcores=NS
)


@functools.partial(
    pl.kernel,
    mesh=_mesh,
    out_type=jax.ShapeDtypeStruct((B,), jnp.float32),
    compiler_params=pltpu.CompilerParams(needs_layout_passes=False),
    scratch_types=[
        pltpu.VMEM((BPW,), jnp.int32),
        pltpu.VMEM((BPW,), jnp.int32),
        pltpu.VMEM((CHUNK, F), jnp.float32),
        pltpu.VMEM((CHUNK, F), jnp.float32),
        pltpu.VMEM((CHUNK, F), jnp.float32),
        pltpu.VMEM((CHUNK, F), jnp.float32),
        pltpu.VMEM((BPW + L,), jnp.float32),
        pltpu.SemaphoreType.DMA,
        pltpu.SemaphoreType.DMA,
        pltpu.SemaphoreType.DMA,
        pltpu.SemaphoreType.DMA,
    ],
)
def _mf_kernel(user_hbm, item_hbm, uf_hbm, if_hbm, out_hbm,
               uidx, iidx, urows0, urows1, vrows0, vrows1, outv,
               semu0, semu1, semv0, semv1):
    wid = lax.axis_index("s") * NC + lax.axis_index("c")
    base = wid * BPW
    pltpu.sync_copy(user_hbm.at[pl.ds(base, BPW)], uidx)
    pltpu.sync_copy(item_hbm.at[pl.ds(base, BPW)], iidx)

    iota = lax.iota(jnp.int32, L)
    ubufs = (urows0, urows1)
    vbufs = (vrows0, vrows1)
    usems = (semu0, semu1)
    vsems = (semv0, semv1)

    def start(c):
        b = c % 2
        cu = pltpu.async_copy(uf_hbm.at[uidx.at[pl.ds(c * CHUNK, CHUNK)]], ubufs[b], usems[b])
        cv = pltpu.async_copy(if_hbm.at[iidx.at[pl.ds(c * CHUNK, CHUNK)]], vbufs[b], vsems[b])
        return cu, cv

    handles = {0: start(0)}
    for c in range(NCHUNK):
        if c + 1 < NCHUNK:
            handles[(c + 1) % 2] = start(c + 1)
        cu, cv = handles[c % 2]
        cu.wait()
        cv.wait()
        urows = ubufs[c % 2]
        vrows = vbufs[c % 2]
        mask_last = iota == (L - 1)

        def do_row(r, carry2, urows=urows, vrows=vrows, c=c):
            acc = urows[r, pl.ds(0, L)] * vrows[r, pl.ds(0, L)]
            for j in range(1, F // L):
                acc = acc + urows[r, pl.ds(j * L, L)] * vrows[r, pl.ds(j * L, L)]
            plsc.store_compressed(outv.at[pl.ds(c * CHUNK + r, L)],
                                  jnp.cumsum(acc), mask=mask_last)
            return carry2

        lax.fori_loop(0, CHUNK, do_row, 0)

    pltpu.sync_copy(outv.at[pl.ds(0, BPW)], out_hbm.at[pl.ds(base, BPW)])


def kernel(user, item, user_factors, item_factors):
    return _mf_kernel(user, item, user_factors, item_factors)


# speedup vs baseline: 1.4406x; 1.0179x over previous
import functools

import jax
import jax.numpy as jnp
from jax import lax
from jax.experimental import pallas as pl
from jax.experimental.pallas import tpu as pltpu
from jax.experimental.pallas import tpu_sc as plsc

NC = 2
NS = 16
L = 16
NW = NC * NS
B = 16384
F = 128
BPW = B // NW
CHUNK = 128
NCHUNK = BPW // CHUNK

_mesh = plsc.VectorSubcoreMesh(
    core_axis_name="c", subcore_axis_name="s", num_cores=NC, num_subcores=NS
)


@functools.partial(
    pl.kernel,
    mesh=_mesh,
    out_type=jax.ShapeDtypeStruct((B,), jnp.float32),
    compiler_params=pltpu.CompilerParams(needs_layout_passes=False),
    scratch_types=[
        pltpu.VMEM((BPW,), jnp.int32),
        pltpu.VMEM((BPW,), jnp.int32),
        pltpu.VMEM((CHUNK, F), jnp.float32),
        pltpu.VMEM((CHUNK, F), jnp.float32),
        pltpu.VMEM((CHUNK, F), jnp.float32),
        pltpu.VMEM((CHUNK, F), jnp.float32),
        pltpu.VMEM((BPW + L,), jnp.float32),
        pltpu.SemaphoreType.DMA,
        pltpu.SemaphoreType.DMA,
        pltpu.SemaphoreType.DMA,
        pltpu.SemaphoreType.DMA,
    ],
)
def _mf_kernel(user_hbm, item_hbm, uf_hbm, if_hbm, out_hbm,
               uidx, iidx, urows0, urows1, vrows0, vrows1, outv,
               semu0, semu1, semv0, semv1):
    wid = lax.axis_index("s") * NC + lax.axis_index("c")
    base = wid * BPW
    ci_u = pltpu.async_copy(user_hbm.at[pl.ds(base, BPW)], uidx, semu1)
    ci_v = pltpu.async_copy(item_hbm.at[pl.ds(base, BPW)], iidx, semv1)
    ci_u.wait()
    ci_v.wait()

    iota = lax.iota(jnp.int32, L)
    ubufs = (urows0, urows1)
    vbufs = (vrows0, vrows1)
    usems = (semu0, semu1)
    vsems = (semv0, semv1)

    def start(c):
        b = c % 2
        cu = pltpu.async_copy(uf_hbm.at[uidx.at[pl.ds(c * CHUNK, CHUNK)]], ubufs[b], usems[b])
        cv = pltpu.async_copy(if_hbm.at[iidx.at[pl.ds(c * CHUNK, CHUNK)]], vbufs[b], vsems[b])
        return cu, cv

    handles = {0: start(0)}
    for c in range(NCHUNK):
        if c + 1 < NCHUNK:
            handles[(c + 1) % 2] = start(c + 1)
        cu, cv = handles[c % 2]
        cu.wait()
        cv.wait()
        urows = ubufs[c % 2]
        vrows = vbufs[c % 2]
        mask_last = iota == (L - 1)

        def do_row(r, carry2, urows=urows, vrows=vrows, c=c):
            acc = urows[r, pl.ds(0, L)] * vrows[r, pl.ds(0, L)]
            for j in range(1, F // L):
                acc = acc + urows[r, pl.ds(j * L, L)] * vrows[r, pl.ds(j * L, L)]
            plsc.store_compressed(outv.at[pl.ds(c * CHUNK + r, L)],
                                  jnp.cumsum(acc), mask=mask_last)
            return carry2

        lax.fori_loop(0, CHUNK, do_row, 0)

    pltpu.sync_copy(outv.at[pl.ds(0, BPW)], out_hbm.at[pl.ds(base, BPW)])


def kernel(user, item, user_factors, item_factors):
    return _mf_kernel(user, item, user_factors, item_factors)
